# Initial kernel scaffold; baseline (speedup 1.0000x reference)
import jax
import jax.numpy as jnp
from jax.experimental import pallas as pl


def kernel(x, edge_index, edge_attr, W_in, b_in, lin_w0, att_src0, att_dst0, lin_edge_w0, att_edge0, bias0, lin_w1, att_src1, att_dst1, lin_edge_w1, att_edge1, bias1, fc1_w, fc1_b, fc2_w, fc2_b, fc3_w, fc3_b):
    raise NotImplementedError("write your pallas kernel here")


# speedup vs baseline: 1.1428x; 1.1428x over previous
import functools
import jax
import jax.numpy as jnp
from jax.experimental import pallas as pl
from jax.experimental.pallas import tpu as pltpu

N = 50000
E = 800000
D_IN = 128
D = 64
H = 4

BLK = 400


def _gelu(v):
    return 0.5 * v * (1.0 + jax.lax.erf(v * 0.7071067811865476))


def _inproj_body(x_ref, w_ref, b_ref, o_ref):
    o_ref[...] = _gelu(
        jnp.dot(x_ref[...], w_ref[...], preferred_element_type=jnp.float32)
        + b_ref[...]
    )


def _inproj(x, w, b):
    return pl.pallas_call(
        _inproj_body,
        grid=(N // BLK,),
        in_specs=[
            pl.BlockSpec((BLK, D_IN), lambda i: (i, 0)),
            pl.BlockSpec((D_IN, D), lambda i: (0, 0)),
            pl.BlockSpec((1, D), lambda i: (0, 0)),
        ],
        out_specs=pl.BlockSpec((BLK, D), lambda i: (i, 0)),
        out_shape=jax.ShapeDtypeStruct((N, D), jnp.float32),
    )(x, w, b.reshape(1, D))


def _proj_body(h_ref, w_ref, asrc_w_ref, adst_w_ref, xh_ref, as_ref, ad_ref):
    xh = jnp.dot(h_ref[...], w_ref[...], preferred_element_type=jnp.float32)
    xh_ref[...] = xh
    prod_s = xh * asrc_w_ref[...]
    prod_d = xh * adst_w_ref[...]
    ps = prod_s.reshape(BLK, H, D).sum(axis=2)
    pd = prod_d.reshape(BLK, H, D).sum(axis=2)
    as_ref[...] = ps
    ad_ref[...] = pd


def _attn_proj(h, lin_w, att_src, att_dst):
    asrc_w = att_src.reshape(1, H * D)
    adst_w = att_dst.reshape(1, H * D)
    return pl.pallas_call(
        _proj_body,
        grid=(N // BLK,),
        in_specs=[
            pl.BlockSpec((BLK, D), lambda i: (i, 0)),
            pl.BlockSpec((D, H * D), lambda i: (0, 0)),
            pl.BlockSpec((1, H * D), lambda i: (0, 0)),
            pl.BlockSpec((1, H * D), lambda i: (0, 0)),
        ],
        out_specs=[
            pl.BlockSpec((BLK, H * D), lambda i: (i, 0)),
            pl.BlockSpec((BLK, H), lambda i: (i, 0)),
            pl.BlockSpec((BLK, H), lambda i: (i, 0)),
        ],
        out_shape=[
            jax.ShapeDtypeStruct((N, H * D), jnp.float32),
            jax.ShapeDtypeStruct((N, H), jnp.float32),
            jax.ShapeDtypeStruct((N, H), jnp.float32),
        ],
    )(h, lin_w, asrc_w, adst_w)


def _mlp_body(h_ref, w1_ref, b1_ref, w2_ref, b2_ref, w3_ref, o_ref):
    i = pl.program_id(0)
    h = _gelu(jnp.dot(h_ref[...], w1_ref[...], preferred_element_type=jnp.float32) + b1_ref[...])
    h = _gelu(jnp.dot(h, w2_ref[...], preferred_element_type=jnp.float32) + b2_ref[...])
    q = jnp.dot(h, w3_ref[...], preferred_element_type=jnp.float32)
    bm = jnp.full((1, 128), jnp.min(q), dtype=jnp.float32)

    @pl.when(i == 0)
    def _():
        o_ref[...] = bm

    @pl.when(i > 0)
    def _():
        o_ref[...] = jnp.minimum(o_ref[...], bm)


def _mlp_min(h, w1, b1, w2, b2, w3, b3):
    w3t = jnp.tile(w3, (1, 128))
    out = pl.pallas_call(
        _mlp_body,
        grid=(N // BLK,),
        in_specs=[
            pl.BlockSpec((BLK, D), lambda i: (i, 0)),
            pl.BlockSpec((D, D), lambda i: (0, 0)),
            pl.BlockSpec((1, D), lambda i: (0, 0)),
            pl.BlockSpec((D, D), lambda i: (0, 0)),
            pl.BlockSpec((1, D), lambda i: (0, 0)),
            pl.BlockSpec((D, 128), lambda i: (0, 0)),
        ],
        out_specs=pl.BlockSpec((1, 128), lambda i: (0, 0)),
        out_shape=jax.ShapeDtypeStruct((1, 128), jnp.float32),
    )(h, w1, b1.reshape(1, D), w2, b2.reshape(1, D), w3t)
    return out[0, :1] + b3


def _gat_layer(h, src, dst, ea, lin_w, att_src, att_dst, lin_edge_w, att_edge, bias):
    xh_flat, a_src, a_dst = _attn_proj(h, lin_w, att_src, att_dst)
    xh = xh_flat.reshape(N, H, D)

    V = (lin_edge_w.reshape(3, H, D) * att_edge.reshape(1, H, D)).sum(axis=2)

    ones = jnp.ones(E, dtype=jnp.float32)
    deg = jax.ops.segment_sum(ones, dst, num_segments=N)
    loop_attr = jax.ops.segment_sum(ea, dst, num_segments=N) / jnp.maximum(deg, 1.0)[:, None]

    q_edge = a_src[src] + ea @ V
    t = a_dst
    q_loop = a_src + loop_attr @ V

    qmax = jax.ops.segment_max(q_edge, dst, num_segments=N)
    qmax = jnp.maximum(qmax, q_loop)
    amax = jax.nn.leaky_relu(qmax + t, negative_slope=0.2)

    alpha_e = jax.nn.leaky_relu(q_edge + t[dst], negative_slope=0.2)
    ex_e = jnp.exp(alpha_e - amax[dst])
    alpha_l = jax.nn.leaky_relu(q_loop + t, negative_slope=0.2)
    ex_l = jnp.exp(alpha_l - amax)

    denom = jax.ops.segment_sum(ex_e, dst, num_segments=N) + ex_l
    out_un = jax.ops.segment_sum(ex_e[:, :, None] * xh[src], dst, num_segments=N)
    out_un = out_un + ex_l[:, :, None] * xh
    out = out_un / (denom[:, :, None] + 1e-16)
    return out.mean(axis=1) + bias


def kernel(x, edge_index, edge_attr, W_in, b_in, lin_w0, att_src0, att_dst0,
           lin_edge_w0, att_edge0, bias0, lin_w1, att_src1, att_dst1,
           lin_edge_w1, att_edge1, bias1, fc1_w, fc1_b, fc2_w, fc2_b, fc3_w, fc3_b):
    src, dst = edge_index[0], edge_index[1]
    ea = edge_attr.at[:, 2].set(1000000.0 / edge_attr[:, 2])

    h = _inproj(x, W_in, b_in)
    h = _gelu(_gat_layer(h, src, dst, ea, lin_w0, att_src0, att_dst0,
                         lin_edge_w0, att_edge0, bias0))
    identity = h
    h = _gat_layer(h, src, dst, ea, lin_w1, att_src1, att_dst1,
                   lin_edge_w1, att_edge1, bias1)
    h = _gelu(h + identity)
    return _mlp_min(h, fc1_w, fc1_b, fc2_w, fc2_b, fc3_w, fc3_b)


# speedup vs baseline: 4.8212x; 4.2188x over previous
import functools
import jax
import jax.numpy as jnp
from jax import lax
from jax.experimental import pallas as pl
from jax.experimental.pallas import tpu as pltpu
from jax.experimental.pallas import tpu_sc as plsc

N = 50000
E = 800000
D_IN = 128
D = 64
H = 4

BLK = 400

NW = 32
CH = 128
NCH = 200
EPW = NCH * CH
EPAD = NW * EPW
STRIPE = 3200


def _sc_mesh():
    return plsc.VectorSubcoreMesh(core_axis_name="c", subcore_axis_name="s")


def _make_flat_scatter_add(AN, M):
    stripe = AN // 16
    assert stripe % 128 == 0
    npt = M // (NW * CH)
    nst = 80
    nblk = npt // nst
    assert npt % nst == 0

    @functools.partial(
        pl.kernel, mesh=_sc_mesh(),
        out_type=jax.ShapeDtypeStruct((2, AN), jnp.float32),
        scratch_types=[
            pltpu.VMEM((nst, CH), jnp.int32),
            pltpu.VMEM((nst, CH), jnp.float32),
            pltpu.VMEM((stripe,), jnp.float32),
            pltpu.VMEM_SHARED((AN,), jnp.float32),
        ],
    )
    def k(idx_hbm, dat_hbm, out_hbm, idx_v, dat_v, buf_v, acc_sh):
        c = lax.axis_index("c")
        s = lax.axis_index("s")
        wid = s * 2 + c
        sbase = s * stripe

        def _fill(i, _):
            buf_v[pl.ds(i * 16, 16)] = jnp.zeros((16,), jnp.float32)
            return 0
        lax.fori_loop(0, stripe // 16, _fill, 0)
        pltpu.sync_copy(buf_v, acc_sh.at[pl.ds(sbase, stripe)])
        plsc.subcore_barrier()

        def _blk(b, _):
            pltpu.sync_copy(idx_hbm.at[pl.ds(wid * npt + b * nst, nst)], idx_v)
            pltpu.sync_copy(dat_hbm.at[pl.ds(wid * npt + b * nst, nst)], dat_v)

            def _chunk(j, _):
                pltpu.sync_copy(dat_v.at[j], acc_sh.at[idx_v.at[j]], add=True)
                return 0
            lax.fori_loop(0, nst, _chunk, 0)
            return 0
        lax.fori_loop(0, nblk, _blk, 0)

        plsc.subcore_barrier()
        pltpu.sync_copy(acc_sh.at[pl.ds(sbase, stripe)], buf_v)
        pltpu.sync_copy(buf_v, out_hbm.at[c].at[pl.ds(sbase, stripe)])

    return k


ANP = 204800
_sc_degsum = _make_flat_scatter_add(ANP, EPAD * 4)

NP = 51200
NST = 40
NBLK = NCH // NST
QR = 12500
QRP = 12800
DUMP = 12700


USE_P1 = False
USE_P2 = False
USE_P3 = False
USE_P4 = True


def _take16(x, idx):
    return x.at[idx].get(mode="promise_in_bounds")


@functools.partial(
    pl.kernel, mesh=_sc_mesh(),
    out_type=jax.ShapeDtypeStruct((H, EPAD), jnp.float32),
    scratch_types=[
        pltpu.VMEM((NP,), jnp.float32),
        pltpu.VMEM((NP,), jnp.float32),
        pltpu.VMEM((NST, CH), jnp.int32),
        pltpu.VMEM((NST, CH), jnp.int32),
        pltpu.VMEM((NST * CH,), jnp.float32),
    ],
)
def _sc_logits(src_hbm, dst_hbm, g_hbm, asrc_hbm, t_hbm, le_hbm,
               asrc_pl, t_pl, src_v, dst_v, g_v):
    c = lax.axis_index("c")
    s = lax.axis_index("s")
    wid = s * 2 + c
    for h in range(H):
        pltpu.sync_copy(asrc_hbm.at[h], asrc_pl)
        pltpu.sync_copy(t_hbm.at[h], t_pl)

        def _blk(b, _):
            pltpu.sync_copy(src_hbm.at[pl.ds(wid * NCH + b * NST, NST)], src_v)
            pltpu.sync_copy(dst_hbm.at[pl.ds(wid * NCH + b * NST, NST)], dst_v)
            pltpu.sync_copy(g_hbm.at[h, pl.ds(wid * EPW + b * NST * CH, NST * CH)], g_v)

            def _row(j, _):
                def _vec(k, _):
                    sv = src_v[j, pl.ds(k * 16, 16)]
                    dv = dst_v[j, pl.ds(k * 16, 16)]
                    g = g_v[pl.ds(j * CH + k * 16, 16)]
                    le = plsc.load_gather(asrc_pl, [sv]) + plsc.load_gather(t_pl, [dv]) + g
                    g_v[pl.ds(j * CH + k * 16, 16)] = jnp.where(le >= 0.0, le, 0.2 * le)
                    return 0
                lax.fori_loop(0, 8, _vec, 0)
                return 0
            lax.fori_loop(0, NST, _row, 0)
            pltpu.sync_copy(g_v, le_hbm.at[h, pl.ds(wid * EPW + b * NST * CH, NST * CH)])
            return 0
        lax.fori_loop(0, NBLK, _blk, 0)


@functools.partial(
    pl.kernel, mesh=_sc_mesh(),
    out_type=jax.ShapeDtypeStruct((2, 2, NP), jnp.float32),
    scratch_types=[
        pltpu.VMEM((NP,), jnp.float32),
        pltpu.VMEM((NP,), jnp.float32),
        pltpu.VMEM((NST, CH), jnp.int32),
        pltpu.VMEM((NST * CH,), jnp.float32),
        pltpu.VMEM((NST * CH,), jnp.float32),
        pltpu.VMEM((3200,), jnp.float32),
        pltpu.VMEM((3200,), jnp.float32),
        pltpu.VMEM_SHARED((32, NP), jnp.float32),
    ],
)
def _sc_segmax(dst_hbm, le0_hbm, le1_hbm, out_hbm,
               tab0, tab1, dst_v, le0_v, le1_v, acc_v, bnc_v, sh):
    c = lax.axis_index("c")
    s = lax.axis_index("s")
    wid = s * 2 + c
    neg = jnp.full((16,), -1e30, jnp.float32)

    def _init(i, _):
        tab0[pl.ds(i * 16, 16)] = neg
        tab1[pl.ds(i * 16, 16)] = neg
        return 0
    lax.fori_loop(0, NP // 16, _init, 0)

    lane = lax.iota(jnp.int32, 16)

    def _blk(b, _):
        pltpu.sync_copy(dst_hbm.at[pl.ds(wid * NCH + b * NST, NST)], dst_v)
        pltpu.sync_copy(le0_hbm.at[pl.ds(wid * EPW + b * NST * CH, NST * CH)], le0_v)
        pltpu.sync_copy(le1_hbm.at[pl.ds(wid * EPW + b * NST * CH, NST * CH)], le1_v)

        def _row(j, _):
            def _vec(k, _):
                dv = dst_v[j, pl.ds(k * 16, 16)]
                l0 = le0_v[pl.ds(j * CH + k * 16, 16)]
                l1 = le1_v[pl.ds(j * CH + k * 16, 16)]
                skey, sval = plsc.sort_key_val(dv, lane)
                p0 = _take16(l0, sval)
                p1 = _take16(l1, sval)
                for d in (1, 2, 4, 8):
                    sh_idx = jnp.maximum(lane - d, 0)
                    ks = _take16(skey, sh_idx)
                    same = skey == ks
                    p0 = jnp.where(same, jnp.maximum(p0, _take16(p0, sh_idx)), p0)
                    p1 = jnp.where(same, jnp.maximum(p1, _take16(p1, sh_idx)), p1)
                nxt = _take16(skey, jnp.minimum(lane + 1, 15))
                is_last = (skey != nxt) | (lane == 15)
                cur0 = plsc.load_gather(tab0, [skey])
                plsc.store_scatter(tab0, [skey], jnp.maximum(p0, cur0), mask=is_last)
                cur1 = plsc.load_gather(tab1, [skey])
                plsc.store_scatter(tab1, [skey], jnp.maximum(p1, cur1), mask=is_last)
                return 0
            lax.fori_loop(0, 8, _vec, 0)
            return 0
        lax.fori_loop(0, NST, _row, 0)
        return 0
    lax.fori_loop(0, NBLK, _blk, 0)

    pltpu.sync_copy(tab0, sh.at[s * 2])
    pltpu.sync_copy(tab1, sh.at[s * 2 + 1])
    plsc.subcore_barrier()

    for h in range(2):
        def _init2(i, _):
            acc_v[pl.ds(i * 16, 16)] = neg
            return 0
        lax.fori_loop(0, 200, _init2, 0)

        def _mrg(t, _):
            pltpu.sync_copy(sh.at[t * 2 + h, pl.ds(s * 3200, 3200)], bnc_v)

            def _mx(i, _):
                acc_v[pl.ds(i * 16, 16)] = jnp.maximum(
                    acc_v[pl.ds(i * 16, 16)], bnc_v[pl.ds(i * 16, 16)])
                return 0
            lax.fori_loop(0, 200, _mx, 0)
            return 0
        lax.fori_loop(0, 16, _mrg, 0)
        pltpu.sync_copy(acc_v, out_hbm.at[c, h, pl.ds(s * 3200, 3200)])


@functools.partial(
    pl.kernel, mesh=_sc_mesh(),
    out_type=jax.ShapeDtypeStruct((H, EPAD), jnp.float32),
    scratch_types=[
        pltpu.VMEM((NP,), jnp.float32),
        pltpu.VMEM((NST, CH), jnp.int32),
        pltpu.VMEM((NST * CH,), jnp.float32),
    ],
)
def _sc_weights(dst_hbm, le_hbm, amax_hbm, w_hbm, amax_pl, dst_v, le_v):
    c = lax.axis_index("c")
    s = lax.axis_index("s")
    wid = s * 2 + c
    for h in range(H):
        pltpu.sync_copy(amax_hbm.at[h], amax_pl)

        def _blk(b, _):
            pltpu.sync_copy(dst_hbm.at[pl.ds(wid * NCH + b * NST, NST)], dst_v)
            pltpu.sync_copy(le_hbm.at[h, pl.ds(wid * EPW + b * NST * CH, NST * CH)], le_v)

            def _row(j, _):
                def _vec(k, _):
                    dv = dst_v[j, pl.ds(k * 16, 16)]
                    le = le_v[pl.ds(j * CH + k * 16, 16)]
                    am = plsc.load_gather(amax_pl, [dv])
                    le_v[pl.ds(j * CH + k * 16, 16)] = jnp.exp(le - am)
                    return 0
                lax.fori_loop(0, 8, _vec, 0)
                return 0
            lax.fori_loop(0, NST, _row, 0)
            pltpu.sync_copy(le_v, w_hbm.at[h, pl.ds(wid * EPW + b * NST * CH, NST * CH)])
            return 0
        lax.fori_loop(0, NBLK, _blk, 0)


def _make_msg(lo):
    NS4 = 8
    NB4 = NCH // NS4

    @functools.partial(
        pl.kernel, mesh=_sc_mesh(),
        out_type=jax.ShapeDtypeStruct((2, QRP, CH), jnp.float32),
        scratch_types=[
            pltpu.VMEM((NS4, CH), jnp.int32),
            pltpu.VMEM((NS4, CH), jnp.int32),
            pltpu.VMEM((NS4 * CH,), jnp.float32),
            pltpu.VMEM((NS4 * CH,), jnp.float32),
            pltpu.VMEM((CH, CH), jnp.float32),
            pltpu.VMEM((1, CH), jnp.int32),
            pltpu.VMEM_SHARED((QRP, CH), jnp.float32),
            pltpu.SemaphoreType.DMA,
        ],
    )
    def k(src_hbm, dst_hbm, xhp_hbm, w0_hbm, w1_hbm, out_hbm,
          src_v, dst_v, w0_v, w1_v, buf, idx1, acc_sh, sema):
        c = lax.axis_index("c")
        s = lax.axis_index("s")
        wid = s * 2 + c

        def _fill(i, _):
            def _f2(k2, _):
                buf[i, pl.ds(k2 * 16, 16)] = jnp.zeros((16,), jnp.float32)
                return 0
            lax.fori_loop(0, 8, _f2, 0)
            return 0
        lax.fori_loop(0, 80, _fill, 0)

        def _z(t, _):
            pltpu.sync_copy(buf.at[pl.ds(0, 80)], acc_sh.at[pl.ds(s * 800 + t * 80, 80)])
            return 0
        lax.fori_loop(0, 10, _z, 0)
        plsc.subcore_barrier()

        def _blk(b, _):
            pltpu.sync_copy(src_hbm.at[pl.ds(wid * NCH + b * NS4, NS4)], src_v)
            pltpu.sync_copy(dst_hbm.at[pl.ds(wid * NCH + b * NS4, NS4)], dst_v)
            pltpu.sync_copy(w0_hbm.at[pl.ds(wid * EPW + b * NS4 * CH, NS4 * CH)], w0_v)
            pltpu.sync_copy(w1_hbm.at[pl.ds(wid * EPW + b * NS4 * CH, NS4 * CH)], w1_v)

            def _chunk(j, _):
                pltpu.async_copy(xhp_hbm.at[src_v.at[j]], buf, sema).wait()

                def _sc_grp(kk, _):
                    w0g = w0_v[pl.ds(j * CH + kk * 16, 16)]
                    w1g = w1_v[pl.ds(j * CH + kk * 16, 16)]
                    for e2 in range(16):
                        row = kk * 16 + e2
                        w0s = w0g[e2]
                        w1s = w1g[e2]
                        for q in range(4):
                            buf[row, pl.ds(q * 16, 16)] = buf[row, pl.ds(q * 16, 16)] * w0s
                        for q in range(4, 8):
                            buf[row, pl.ds(q * 16, 16)] = buf[row, pl.ds(q * 16, 16)] * w1s
                    return 0
                lax.fori_loop(0, 8, _sc_grp, 0)

                def _idx(kk, _):
                    dv = dst_v[j, pl.ds(kk * 16, 16)]
                    inq = (dv >= lo) & (dv < lo + QR)
                    idx1[0, pl.ds(kk * 16, 16)] = jnp.where(inq, dv - lo, DUMP)
                    return 0
                lax.fori_loop(0, 8, _idx, 0)
                pltpu.sync_copy(buf, acc_sh.at[idx1.at[0]], add=True)
                return 0
            lax.fori_loop(0, NS4, _chunk, 0)
            return 0
        lax.fori_loop(0, NB4, _blk, 0)

        plsc.subcore_barrier()

        def _dump(t, _):
            pltpu.sync_copy(acc_sh.at[pl.ds(s * 800 + t * 80, 80)], buf.at[pl.ds(0, 80)])
            pltpu.sync_copy(buf.at[pl.ds(0, 80)], out_hbm.at[c].at[pl.ds(s * 800 + t * 80, 80)])
            return 0
        lax.fori_loop(0, 10, _dump, 0)

    return k


_sc_msg = [_make_msg(r * QR) for r in range(4)]


def _gelu(v):
    return 0.5 * v * (1.0 + jax.lax.erf(v * 0.7071067811865476))


def _inproj_body(x_ref, w_ref, b_ref, o_ref):
    o_ref[...] = _gelu(
        jnp.dot(x_ref[...], w_ref[...], preferred_element_type=jnp.float32)
        + b_ref[...]
    )


def _inproj(x, w, b):
    return pl.pallas_call(
        _inproj_body,
        grid=(N // BLK,),
        in_specs=[
            pl.BlockSpec((BLK, D_IN), lambda i: (i, 0)),
            pl.BlockSpec((D_IN, D), lambda i: (0, 0)),
            pl.BlockSpec((1, D), lambda i: (0, 0)),
        ],
        out_specs=pl.BlockSpec((BLK, D), lambda i: (i, 0)),
        out_shape=jax.ShapeDtypeStruct((N, D), jnp.float32),
    )(x, w, b.reshape(1, D))


def _proj_body(h_ref, w_ref, asrc_w_ref, adst_w_ref, xh_ref, as_ref, ad_ref):
    xh = jnp.dot(h_ref[...], w_ref[...], preferred_element_type=jnp.float32)
    xh_ref[...] = xh
    prod_s = xh * asrc_w_ref[...]
    prod_d = xh * adst_w_ref[...]
    ps = prod_s.reshape(BLK, H, D).sum(axis=2)
    pd = prod_d.reshape(BLK, H, D).sum(axis=2)
    as_ref[...] = ps
    ad_ref[...] = pd


def _attn_proj(h, lin_w, att_src, att_dst):
    asrc_w = att_src.reshape(1, H * D)
    adst_w = att_dst.reshape(1, H * D)
    return pl.pallas_call(
        _proj_body,
        grid=(N // BLK,),
        in_specs=[
            pl.BlockSpec((BLK, D), lambda i: (i, 0)),
            pl.BlockSpec((D, H * D), lambda i: (0, 0)),
            pl.BlockSpec((1, H * D), lambda i: (0, 0)),
            pl.BlockSpec((1, H * D), lambda i: (0, 0)),
        ],
        out_specs=[
            pl.BlockSpec((BLK, H * D), lambda i: (i, 0)),
            pl.BlockSpec((BLK, H), lambda i: (i, 0)),
            pl.BlockSpec((BLK, H), lambda i: (i, 0)),
        ],
        out_shape=[
            jax.ShapeDtypeStruct((N, H * D), jnp.float32),
            jax.ShapeDtypeStruct((N, H), jnp.float32),
            jax.ShapeDtypeStruct((N, H), jnp.float32),
        ],
    )(h, lin_w, asrc_w, adst_w)


def _mlp_body(h_ref, w1_ref, b1_ref, w2_ref, b2_ref, w3_ref, o_ref):
    i = pl.program_id(0)
    h = _gelu(jnp.dot(h_ref[...], w1_ref[...], preferred_element_type=jnp.float32) + b1_ref[...])
    h = _gelu(jnp.dot(h, w2_ref[...], preferred_element_type=jnp.float32) + b2_ref[...])
    q = jnp.dot(h, w3_ref[...], preferred_element_type=jnp.float32)
    bm = jnp.full((1, 128), jnp.min(q), dtype=jnp.float32)

    @pl.when(i == 0)
    def _():
        o_ref[...] = bm

    @pl.when(i > 0)
    def _():
        o_ref[...] = jnp.minimum(o_ref[...], bm)


def _mlp_min(h, w1, b1, w2, b2, w3, b3):
    w3t = jnp.tile(w3, (1, 128))
    out = pl.pallas_call(
        _mlp_body,
        grid=(N // BLK,),
        in_specs=[
            pl.BlockSpec((BLK, D), lambda i: (i, 0)),
            pl.BlockSpec((D, D), lambda i: (0, 0)),
            pl.BlockSpec((1, D), lambda i: (0, 0)),
            pl.BlockSpec((D, D), lambda i: (0, 0)),
            pl.BlockSpec((1, D), lambda i: (0, 0)),
            pl.BlockSpec((D, 128), lambda i: (0, 0)),
        ],
        out_specs=pl.BlockSpec((1, 128), lambda i: (0, 0)),
        out_shape=jax.ShapeDtypeStruct((1, 128), jnp.float32),
    )(h, w1, b1.reshape(1, D), w2, b2.reshape(1, D), w3t)
    return out[0, :1] + b3


def _pad_planes(a):
    return jnp.pad(a.T, ((0, 0), (0, NP - N)))


def _gat_layer(h, srcb, dstb, src_ids, dst, idxA, ea, loop_attr,
               lin_w, att_src, att_dst, lin_edge_w, att_edge, bias):
    xh_flat, a_src, a_dst = _attn_proj(h, lin_w, att_src, att_dst)
    xh = xh_flat.reshape(N, H, D)
    t = a_dst

    V = (lin_edge_w.reshape(3, H, D) * att_edge.reshape(1, H, D)).sum(axis=2)

    g = ea @ V
    gT = jnp.concatenate(
        [g.T, jnp.full((H, EPAD - E), -1e30, jnp.float32)], axis=1)

    if USE_P1:
        leT = _sc_logits(srcb, dstb, gT, _pad_planes(a_src), _pad_planes(t))
    else:
        le = jax.nn.leaky_relu(a_src[src_ids] + t[dst] + g, negative_slope=0.2)
        leT = jnp.concatenate(
            [le.T, jnp.full((H, EPAD - E), -2e29, jnp.float32)], axis=1)

    if USE_P2:
        m01 = _sc_segmax(dstb, leT[0], leT[1])
        m23 = _sc_segmax(dstb, leT[2], leT[3])
        le_max = jnp.concatenate([jnp.max(m01, axis=0), jnp.max(m23, axis=0)],
                                 axis=0)[:, :N].T
    else:
        le_max = jax.ops.segment_max(
            leT[:, :E].T, dst, num_segments=N)
        le_max = jnp.maximum(le_max, -1e30)

    q_loop = a_src + loop_attr @ V
    alpha_l = jax.nn.leaky_relu(q_loop + t, negative_slope=0.2)
    amax = jnp.maximum(le_max, alpha_l)

    if USE_P3:
        wT = _sc_weights(dstb, leT, _pad_planes(amax))
    else:
        wT = jnp.exp(leT - _pad_planes(amax)[:, :1] * 0.0
                     - jnp.concatenate([amax.T[:, dst],
                                        jnp.zeros((H, EPAD - E), jnp.float32)], axis=1))

    ex_l = jnp.exp(alpha_l - amax)

    datD = wT[:, :E].T.reshape(-1)
    datD = jnp.concatenate([datD, jnp.zeros(4 * (EPAD - E), jnp.float32)])
    dparts = _sc_degsum(idxA.reshape(-1, CH), datD.reshape(-1, CH))
    denom = (dparts[0, :4 * N] + dparts[1, :4 * N]).reshape(N, 4) + ex_l

    if USE_P4:
        out_cols = []
        for hp in range(2):
            xhp = xh_flat[:, hp * 128:(hp + 1) * 128]
            w0 = wT[2 * hp]
            w1 = wT[2 * hp + 1]
            rows = []
            for r in range(4):
                part = _sc_msg[r](srcb, dstb, xhp, w0, w1)
                rows.append((part[0] + part[1])[:QR])
            out_cols.append(jnp.concatenate(rows, axis=0)[:N])
        out_un = jnp.concatenate(out_cols, axis=1).reshape(N, H, D)
    else:
        ex_e = wT[:, :E].T
        out_un = jax.ops.segment_sum(ex_e[:, :, None] * xh[src_ids], dst,
                                     num_segments=N)
    out_un = out_un + ex_l[:, :, None] * xh
    out = out_un / (denom[:, :, None] + 1e-16)
    return out.mean(axis=1) + bias


def kernel(x, edge_index, edge_attr, W_in, b_in, lin_w0, att_src0, att_dst0,
           lin_edge_w0, att_edge0, bias0, lin_w1, att_src1, att_dst1,
           lin_edge_w1, att_edge1, bias1, fc1_w, fc1_b, fc2_w, fc2_b, fc3_w, fc3_b):
    src, dst = edge_index[0], edge_index[1]
    ea = edge_attr.at[:, 2].set(1000000.0 / edge_attr[:, 2])

    idxA = (4 * dst[:, None] + jnp.arange(4, dtype=jnp.int32)[None, :]).reshape(-1)
    idxA = jnp.concatenate([idxA, jnp.zeros(4 * (EPAD - E), jnp.int32)])
    datA = jnp.concatenate([jnp.ones((E, 1), jnp.float32), ea], axis=1).reshape(-1)
    datA = jnp.concatenate([datA, jnp.zeros(4 * (EPAD - E), jnp.float32)])
    parts = _sc_degsum(idxA.reshape(-1, CH), datA.reshape(-1, CH))
    merged = (parts[0, :4 * N] + parts[1, :4 * N]).reshape(N, 4)
    deg = merged[:, 0]
    loop_attr = merged[:, 1:4] / jnp.maximum(deg, 1.0)[:, None]

    srcb = jnp.concatenate([src, jnp.zeros(EPAD - E, jnp.int32)]).reshape(EPAD // CH, CH)
    dstb = jnp.concatenate([dst, jnp.zeros(EPAD - E, jnp.int32)]).reshape(EPAD // CH, CH)

    h = _inproj(x, W_in, b_in)
    h = _gelu(_gat_layer(h, srcb, dstb, src, dst, idxA, ea, loop_attr,
                         lin_w0, att_src0, att_dst0, lin_edge_w0, att_edge0, bias0))
    identity = h
    h = _gat_layer(h, srcb, dstb, src, dst, idxA, ea, loop_attr,
                   lin_w1, att_src1, att_dst1, lin_edge_w1, att_edge1, bias1)
    h = _gelu(h + identity)
    return _mlp_min(h, fc1_w, fc1_b, fc2_w, fc2_b, fc3_w, fc3_b)


# speedup vs baseline: 5.9210x; 1.2281x over previous
import functools
import jax
import jax.numpy as jnp
from jax import lax
from jax.experimental import pallas as pl
from jax.experimental.pallas import tpu as pltpu
from jax.experimental.pallas import tpu_sc as plsc

N = 50000
E = 800000
D_IN = 128
D = 64
H = 4

BLK = 400

NW = 32
CH = 128
NCH = 200
EPW = NCH * CH
EPAD = NW * EPW
STRIPE = 3200


def _sc_mesh():
    return plsc.VectorSubcoreMesh(core_axis_name="c", subcore_axis_name="s")


def _make_flat_scatter_add(AN, M):
    stripe = AN // 16
    assert stripe % 128 == 0
    npt = M // (NW * CH)
    nst = 80
    nblk = npt // nst
    assert npt % nst == 0

    @functools.partial(
        pl.kernel, mesh=_sc_mesh(),
        out_type=jax.ShapeDtypeStruct((2, AN), jnp.float32),
        scratch_types=[
            pltpu.VMEM((nst, CH), jnp.int32),
            pltpu.VMEM((nst, CH), jnp.float32),
            pltpu.VMEM((stripe,), jnp.float32),
            pltpu.VMEM_SHARED((AN,), jnp.float32),
        ],
    )
    def k(idx_hbm, dat_hbm, out_hbm, idx_v, dat_v, buf_v, acc_sh):
        c = lax.axis_index("c")
        s = lax.axis_index("s")
        wid = s * 2 + c
        sbase = s * stripe

        def _fill(i, _):
            buf_v[pl.ds(i * 16, 16)] = jnp.zeros((16,), jnp.float32)
            return 0
        lax.fori_loop(0, stripe // 16, _fill, 0)
        pltpu.sync_copy(buf_v, acc_sh.at[pl.ds(sbase, stripe)])
        plsc.subcore_barrier()

        def _blk(b, _):
            pltpu.sync_copy(idx_hbm.at[pl.ds(wid * npt + b * nst, nst)], idx_v)
            pltpu.sync_copy(dat_hbm.at[pl.ds(wid * npt + b * nst, nst)], dat_v)

            def _chunk(j, _):
                pltpu.sync_copy(dat_v.at[j], acc_sh.at[idx_v.at[j]], add=True)
                return 0
            lax.fori_loop(0, nst, _chunk, 0)
            return 0
        lax.fori_loop(0, nblk, _blk, 0)

        plsc.subcore_barrier()
        pltpu.sync_copy(acc_sh.at[pl.ds(sbase, stripe)], buf_v)
        pltpu.sync_copy(buf_v, out_hbm.at[c].at[pl.ds(sbase, stripe)])

    return k


ANP = 204800
_sc_degsum = _make_flat_scatter_add(ANP, EPAD * 4)

NP = 51200
NST = 40
NBLK = NCH // NST
QR = 12500
QRP = 12800
DUMP = 12700


G1B = 8
G1NB = (EPAD * 4) // (NW * CH * G1B)


@functools.partial(
    pl.kernel, mesh=_sc_mesh(),
    out_type=jax.ShapeDtypeStruct((EPAD * 4,), jnp.float32),
    scratch_types=[
        pltpu.VMEM((G1B, CH), jnp.int32),
        pltpu.VMEM((G1B, CH), jnp.int32),
        pltpu.VMEM((G1B * CH,), jnp.float32),
        pltpu.VMEM((CH,), jnp.float32),
        pltpu.VMEM((CH,), jnp.float32),
        pltpu.VMEM((ANP // 16,), jnp.float32),
        pltpu.VMEM_SHARED((ANP,), jnp.float32),
        pltpu.VMEM_SHARED((ANP,), jnp.float32),
    ],
)
def _sc_le(idxs_hbm, idxd_hbm, g_hbm, asrc_hbm, t_hbm, le_hbm,
           idxs_v, idxd_v, g_v, bA, bB, stg, shA, shB):
    c = lax.axis_index("c")
    s = lax.axis_index("s")
    wid = s * 2 + c
    npt = (EPAD * 4) // (NW * CH)
    stripe = ANP // 16
    pltpu.sync_copy(asrc_hbm.at[pl.ds(s * stripe, stripe)], stg)
    pltpu.sync_copy(stg, shA.at[pl.ds(s * stripe, stripe)])
    pltpu.sync_copy(t_hbm.at[pl.ds(s * stripe, stripe)], stg)
    pltpu.sync_copy(stg, shB.at[pl.ds(s * stripe, stripe)])
    plsc.subcore_barrier()

    def _blk(b, _):
        base = wid * npt + b * G1B
        pltpu.sync_copy(idxs_hbm.at[pl.ds(base, G1B)], idxs_v)
        pltpu.sync_copy(idxd_hbm.at[pl.ds(base, G1B)], idxd_v)
        pltpu.sync_copy(g_hbm.at[pl.ds(base * CH, G1B * CH)], g_v)

        def _chunk(j, _):
            pltpu.sync_copy(shA.at[idxs_v.at[j]], bA)
            pltpu.sync_copy(shB.at[idxd_v.at[j]], bB)

            def _vec(k, _):
                le = bA[pl.ds(k * 16, 16)] + bB[pl.ds(k * 16, 16)] \
                    + g_v[pl.ds(j * CH + k * 16, 16)]
                g_v[pl.ds(j * CH + k * 16, 16)] = jnp.where(le >= 0.0, le, 0.2 * le)
                return 0
            lax.fori_loop(0, 8, _vec, 0)
            return 0
        lax.fori_loop(0, G1B, _chunk, 0)
        pltpu.sync_copy(g_v, le_hbm.at[pl.ds(base * CH, G1B * CH)])
        return 0
    lax.fori_loop(0, G1NB, _blk, 0)


@functools.partial(
    pl.kernel, mesh=_sc_mesh(),
    out_type=jax.ShapeDtypeStruct((EPAD * 4,), jnp.float32),
    scratch_types=[
        pltpu.VMEM((G1B, CH), jnp.int32),
        pltpu.VMEM((G1B * CH,), jnp.float32),
        pltpu.VMEM((CH,), jnp.float32),
        pltpu.VMEM((ANP // 16,), jnp.float32),
        pltpu.VMEM_SHARED((ANP,), jnp.float32),
    ],
)
def _sc_w(idxd_hbm, le_hbm, amax_hbm, w_hbm, idxd_v, le_v, bA, stg, shA):
    c = lax.axis_index("c")
    s = lax.axis_index("s")
    wid = s * 2 + c
    npt = (EPAD * 4) // (NW * CH)
    stripe = ANP // 16
    pltpu.sync_copy(amax_hbm.at[pl.ds(s * stripe, stripe)], stg)
    pltpu.sync_copy(stg, shA.at[pl.ds(s * stripe, stripe)])
    plsc.subcore_barrier()

    def _blk(b, _):
        base = wid * npt + b * G1B
        pltpu.sync_copy(idxd_hbm.at[pl.ds(base, G1B)], idxd_v)
        pltpu.sync_copy(le_hbm.at[pl.ds(base * CH, G1B * CH)], le_v)

        def _chunk(j, _):
            pltpu.sync_copy(shA.at[idxd_v.at[j]], bA)

            def _vec(k, _):
                le = le_v[pl.ds(j * CH + k * 16, 16)]
                le_v[pl.ds(j * CH + k * 16, 16)] = jnp.exp(le - bA[pl.ds(k * 16, 16)])
                return 0
            lax.fori_loop(0, 8, _vec, 0)
            return 0
        lax.fori_loop(0, G1B, _chunk, 0)
        pltpu.sync_copy(le_v, w_hbm.at[pl.ds(base * CH, G1B * CH)])
        return 0
    lax.fori_loop(0, G1NB, _blk, 0)


USE_P1 = False
USE_P2 = False
USE_P3 = False
USE_P4 = True


def _take16(x, idx):
    return x.at[idx].get(mode="promise_in_bounds")


@functools.partial(
    pl.kernel, mesh=_sc_mesh(),
    out_type=jax.ShapeDtypeStruct((H, EPAD), jnp.float32),
    scratch_types=[
        pltpu.VMEM((NP,), jnp.float32),
        pltpu.VMEM((NP,), jnp.float32),
        pltpu.VMEM((NST, CH), jnp.int32),
        pltpu.VMEM((NST, CH), jnp.int32),
        pltpu.VMEM((NST * CH,), jnp.float32),
    ],
)
def _sc_logits(src_hbm, dst_hbm, g_hbm, asrc_hbm, t_hbm, le_hbm,
               asrc_pl, t_pl, src_v, dst_v, g_v):
    c = lax.axis_index("c")
    s = lax.axis_index("s")
    wid = s * 2 + c
    for h in range(H):
        pltpu.sync_copy(asrc_hbm.at[h], asrc_pl)
        pltpu.sync_copy(t_hbm.at[h], t_pl)

        def _blk(b, _):
            pltpu.sync_copy(src_hbm.at[pl.ds(wid * NCH + b * NST, NST)], src_v)
            pltpu.sync_copy(dst_hbm.at[pl.ds(wid * NCH + b * NST, NST)], dst_v)
            pltpu.sync_copy(g_hbm.at[h, pl.ds(wid * EPW + b * NST * CH, NST * CH)], g_v)

            def _row(j, _):
                def _vec(k, _):
                    sv = src_v[j, pl.ds(k * 16, 16)]
                    dv = dst_v[j, pl.ds(k * 16, 16)]
                    g = g_v[pl.ds(j * CH + k * 16, 16)]
                    le = plsc.load_gather(asrc_pl, [sv]) + plsc.load_gather(t_pl, [dv]) + g
                    g_v[pl.ds(j * CH + k * 16, 16)] = jnp.where(le >= 0.0, le, 0.2 * le)
                    return 0
                lax.fori_loop(0, 8, _vec, 0)
                return 0
            lax.fori_loop(0, NST, _row, 0)
            pltpu.sync_copy(g_v, le_hbm.at[h, pl.ds(wid * EPW + b * NST * CH, NST * CH)])
            return 0
        lax.fori_loop(0, NBLK, _blk, 0)


@functools.partial(
    pl.kernel, mesh=_sc_mesh(),
    out_type=jax.ShapeDtypeStruct((2, 2, NP), jnp.float32),
    scratch_types=[
        pltpu.VMEM((NP,), jnp.float32),
        pltpu.VMEM((NP,), jnp.float32),
        pltpu.VMEM((NST, CH), jnp.int32),
        pltpu.VMEM((NST * CH,), jnp.float32),
        pltpu.VMEM((NST * CH,), jnp.float32),
        pltpu.VMEM((3200,), jnp.float32),
        pltpu.VMEM((3200,), jnp.float32),
        pltpu.VMEM_SHARED((32, NP), jnp.float32),
    ],
)
def _sc_segmax(dst_hbm, le0_hbm, le1_hbm, out_hbm,
               tab0, tab1, dst_v, le0_v, le1_v, acc_v, bnc_v, sh):
    c = lax.axis_index("c")
    s = lax.axis_index("s")
    wid = s * 2 + c
    neg = jnp.full((16,), -1e30, jnp.float32)

    def _init(i, _):
        tab0[pl.ds(i * 16, 16)] = neg
        tab1[pl.ds(i * 16, 16)] = neg
        return 0
    lax.fori_loop(0, NP // 16, _init, 0)

    lane = lax.iota(jnp.int32, 16)

    def _blk(b, _):
        pltpu.sync_copy(dst_hbm.at[pl.ds(wid * NCH + b * NST, NST)], dst_v)
        pltpu.sync_copy(le0_hbm.at[pl.ds(wid * EPW + b * NST * CH, NST * CH)], le0_v)
        pltpu.sync_copy(le1_hbm.at[pl.ds(wid * EPW + b * NST * CH, NST * CH)], le1_v)

        def _row(j, _):
            def _vec(k, _):
                dv = dst_v[j, pl.ds(k * 16, 16)]
                l0 = le0_v[pl.ds(j * CH + k * 16, 16)]
                l1 = le1_v[pl.ds(j * CH + k * 16, 16)]
                skey, sval = plsc.sort_key_val(dv, lane)
                p0 = _take16(l0, sval)
                p1 = _take16(l1, sval)
                for d in (1, 2, 4, 8):
                    sh_idx = jnp.maximum(lane - d, 0)
                    ks = _take16(skey, sh_idx)
                    same = skey == ks
                    p0 = jnp.where(same, jnp.maximum(p0, _take16(p0, sh_idx)), p0)
                    p1 = jnp.where(same, jnp.maximum(p1, _take16(p1, sh_idx)), p1)
                nxt = _take16(skey, jnp.minimum(lane + 1, 15))
                is_last = (skey != nxt) | (lane == 15)
                cur0 = plsc.load_gather(tab0, [skey])
                plsc.store_scatter(tab0, [skey], jnp.maximum(p0, cur0), mask=is_last)
                cur1 = plsc.load_gather(tab1, [skey])
                plsc.store_scatter(tab1, [skey], jnp.maximum(p1, cur1), mask=is_last)
                return 0
            lax.fori_loop(0, 8, _vec, 0)
            return 0
        lax.fori_loop(0, NST, _row, 0)
        return 0
    lax.fori_loop(0, NBLK, _blk, 0)

    pltpu.sync_copy(tab0, sh.at[s * 2])
    pltpu.sync_copy(tab1, sh.at[s * 2 + 1])
    plsc.subcore_barrier()

    for h in range(2):
        def _init2(i, _):
            acc_v[pl.ds(i * 16, 16)] = neg
            return 0
        lax.fori_loop(0, 200, _init2, 0)

        def _mrg(t, _):
            pltpu.sync_copy(sh.at[t * 2 + h, pl.ds(s * 3200, 3200)], bnc_v)

            def _mx(i, _):
                acc_v[pl.ds(i * 16, 16)] = jnp.maximum(
                    acc_v[pl.ds(i * 16, 16)], bnc_v[pl.ds(i * 16, 16)])
                return 0
            lax.fori_loop(0, 200, _mx, 0)
            return 0
        lax.fori_loop(0, 16, _mrg, 0)
        pltpu.sync_copy(acc_v, out_hbm.at[c, h, pl.ds(s * 3200, 3200)])


@functools.partial(
    pl.kernel, mesh=_sc_mesh(),
    out_type=jax.ShapeDtypeStruct((H, EPAD), jnp.float32),
    scratch_types=[
        pltpu.VMEM((NP,), jnp.float32),
        pltpu.VMEM((NST, CH), jnp.int32),
        pltpu.VMEM((NST * CH,), jnp.float32),
    ],
)
def _sc_weights(dst_hbm, le_hbm, amax_hbm, w_hbm, amax_pl, dst_v, le_v):
    c = lax.axis_index("c")
    s = lax.axis_index("s")
    wid = s * 2 + c
    for h in range(H):
        pltpu.sync_copy(amax_hbm.at[h], amax_pl)

        def _blk(b, _):
            pltpu.sync_copy(dst_hbm.at[pl.ds(wid * NCH + b * NST, NST)], dst_v)
            pltpu.sync_copy(le_hbm.at[h, pl.ds(wid * EPW + b * NST * CH, NST * CH)], le_v)

            def _row(j, _):
                def _vec(k, _):
                    dv = dst_v[j, pl.ds(k * 16, 16)]
                    le = le_v[pl.ds(j * CH + k * 16, 16)]
                    am = plsc.load_gather(amax_pl, [dv])
                    le_v[pl.ds(j * CH + k * 16, 16)] = jnp.exp(le - am)
                    return 0
                lax.fori_loop(0, 8, _vec, 0)
                return 0
            lax.fori_loop(0, NST, _row, 0)
            pltpu.sync_copy(le_v, w_hbm.at[h, pl.ds(wid * EPW + b * NST * CH, NST * CH)])
            return 0
        lax.fori_loop(0, NBLK, _blk, 0)


def _make_msg(lo):
    NS4 = 8
    NB4 = NCH // NS4

    @functools.partial(
        pl.kernel, mesh=_sc_mesh(),
        out_type=jax.ShapeDtypeStruct((2, QRP, CH), jnp.float32),
        scratch_types=[
            pltpu.VMEM((NS4, CH), jnp.int32),
            pltpu.VMEM((NS4, CH), jnp.int32),
            pltpu.VMEM((NS4 * CH,), jnp.float32),
            pltpu.VMEM((NS4 * CH,), jnp.float32),
            pltpu.VMEM((CH, CH), jnp.float32),
            pltpu.VMEM((1, CH), jnp.int32),
            pltpu.VMEM_SHARED((QRP, CH), jnp.float32),
            pltpu.SemaphoreType.DMA,
        ],
    )
    def k(src_hbm, dst_hbm, xhp_hbm, w0_hbm, w1_hbm, out_hbm,
          src_v, dst_v, w0_v, w1_v, buf, idx1, acc_sh, sema):
        c = lax.axis_index("c")
        s = lax.axis_index("s")
        wid = s * 2 + c

        def _fill(i, _):
            def _f2(k2, _):
                buf[i, pl.ds(k2 * 16, 16)] = jnp.zeros((16,), jnp.float32)
                return 0
            lax.fori_loop(0, 8, _f2, 0)
            return 0
        lax.fori_loop(0, 80, _fill, 0)

        def _z(t, _):
            pltpu.sync_copy(buf.at[pl.ds(0, 80)], acc_sh.at[pl.ds(s * 800 + t * 80, 80)])
            return 0
        lax.fori_loop(0, 10, _z, 0)
        plsc.subcore_barrier()

        def _blk(b, _):
            pltpu.sync_copy(src_hbm.at[pl.ds(wid * NCH + b * NS4, NS4)], src_v)
            pltpu.sync_copy(dst_hbm.at[pl.ds(wid * NCH + b * NS4, NS4)], dst_v)
            pltpu.sync_copy(w0_hbm.at[pl.ds(wid * EPW + b * NS4 * CH, NS4 * CH)], w0_v)
            pltpu.sync_copy(w1_hbm.at[pl.ds(wid * EPW + b * NS4 * CH, NS4 * CH)], w1_v)

            def _chunk(j, _):
                pltpu.async_copy(xhp_hbm.at[src_v.at[j]], buf, sema).wait()

                def _sc_grp(kk, _):
                    w0g = w0_v[pl.ds(j * CH + kk * 16, 16)]
                    w1g = w1_v[pl.ds(j * CH + kk * 16, 16)]
                    for e2 in range(16):
                        row = kk * 16 + e2
                        w0s = w0g[e2]
                        w1s = w1g[e2]
                        for q in range(4):
                            buf[row, pl.ds(q * 16, 16)] = buf[row, pl.ds(q * 16, 16)] * w0s
                        for q in range(4, 8):
                            buf[row, pl.ds(q * 16, 16)] = buf[row, pl.ds(q * 16, 16)] * w1s
                    return 0
                lax.fori_loop(0, 8, _sc_grp, 0)

                def _idx(kk, _):
                    dv = dst_v[j, pl.ds(kk * 16, 16)]
                    inq = (dv >= lo) & (dv < lo + QR)
                    idx1[0, pl.ds(kk * 16, 16)] = jnp.where(inq, dv - lo, DUMP)
                    return 0
                lax.fori_loop(0, 8, _idx, 0)
                pltpu.sync_copy(buf, acc_sh.at[idx1.at[0]], add=True)
                return 0
            lax.fori_loop(0, NS4, _chunk, 0)
            return 0
        lax.fori_loop(0, NB4, _blk, 0)

        plsc.subcore_barrier()

        def _dump(t, _):
            pltpu.sync_copy(acc_sh.at[pl.ds(s * 800 + t * 80, 80)], buf.at[pl.ds(0, 80)])
            pltpu.sync_copy(buf.at[pl.ds(0, 80)], out_hbm.at[c].at[pl.ds(s * 800 + t * 80, 80)])
            return 0
        lax.fori_loop(0, 10, _dump, 0)

    return k


_sc_msg = [_make_msg(r * QR) for r in range(4)]


def _gelu(v):
    return 0.5 * v * (1.0 + jax.lax.erf(v * 0.7071067811865476))


def _inproj_body(x_ref, w_ref, b_ref, o_ref):
    o_ref[...] = _gelu(
        jnp.dot(x_ref[...], w_ref[...], preferred_element_type=jnp.float32)
        + b_ref[...]
    )


def _inproj(x, w, b):
    return pl.pallas_call(
        _inproj_body,
        grid=(N // BLK,),
        in_specs=[
            pl.BlockSpec((BLK, D_IN), lambda i: (i, 0)),
            pl.BlockSpec((D_IN, D), lambda i: (0, 0)),
            pl.BlockSpec((1, D), lambda i: (0, 0)),
        ],
        out_specs=pl.BlockSpec((BLK, D), lambda i: (i, 0)),
        out_shape=jax.ShapeDtypeStruct((N, D), jnp.float32),
    )(x, w, b.reshape(1, D))


def _proj_body(h_ref, w_ref, asrc_w_ref, adst_w_ref, xh_ref, as_ref, ad_ref):
    xh = jnp.dot(h_ref[...], w_ref[...], preferred_element_type=jnp.float32)
    xh_ref[...] = xh
    prod_s = xh * asrc_w_ref[...]
    prod_d = xh * adst_w_ref[...]
    ps = prod_s.reshape(BLK, H, D).sum(axis=2)
    pd = prod_d.reshape(BLK, H, D).sum(axis=2)
    as_ref[...] = ps
    ad_ref[...] = pd


def _attn_proj(h, lin_w, att_src, att_dst):
    asrc_w = att_src.reshape(1, H * D)
    adst_w = att_dst.reshape(1, H * D)
    return pl.pallas_call(
        _proj_body,
        grid=(N // BLK,),
        in_specs=[
            pl.BlockSpec((BLK, D), lambda i: (i, 0)),
            pl.BlockSpec((D, H * D), lambda i: (0, 0)),
            pl.BlockSpec((1, H * D), lambda i: (0, 0)),
            pl.BlockSpec((1, H * D), lambda i: (0, 0)),
        ],
        out_specs=[
            pl.BlockSpec((BLK, H * D), lambda i: (i, 0)),
            pl.BlockSpec((BLK, H), lambda i: (i, 0)),
            pl.BlockSpec((BLK, H), lambda i: (i, 0)),
        ],
        out_shape=[
            jax.ShapeDtypeStruct((N, H * D), jnp.float32),
            jax.ShapeDtypeStruct((N, H), jnp.float32),
            jax.ShapeDtypeStruct((N, H), jnp.float32),
        ],
    )(h, lin_w, asrc_w, adst_w)


def _mlp_body(h_ref, w1_ref, b1_ref, w2_ref, b2_ref, w3_ref, o_ref):
    i = pl.program_id(0)
    h = _gelu(jnp.dot(h_ref[...], w1_ref[...], preferred_element_type=jnp.float32) + b1_ref[...])
    h = _gelu(jnp.dot(h, w2_ref[...], preferred_element_type=jnp.float32) + b2_ref[...])
    q = jnp.dot(h, w3_ref[...], preferred_element_type=jnp.float32)
    bm = jnp.full((1, 128), jnp.min(q), dtype=jnp.float32)

    @pl.when(i == 0)
    def _():
        o_ref[...] = bm

    @pl.when(i > 0)
    def _():
        o_ref[...] = jnp.minimum(o_ref[...], bm)


def _mlp_min(h, w1, b1, w2, b2, w3, b3):
    w3t = jnp.tile(w3, (1, 128))
    out = pl.pallas_call(
        _mlp_body,
        grid=(N // BLK,),
        in_specs=[
            pl.BlockSpec((BLK, D), lambda i: (i, 0)),
            pl.BlockSpec((D, D), lambda i: (0, 0)),
            pl.BlockSpec((1, D), lambda i: (0, 0)),
            pl.BlockSpec((D, D), lambda i: (0, 0)),
            pl.BlockSpec((1, D), lambda i: (0, 0)),
            pl.BlockSpec((D, 128), lambda i: (0, 0)),
        ],
        out_specs=pl.BlockSpec((1, 128), lambda i: (0, 0)),
        out_shape=jax.ShapeDtypeStruct((1, 128), jnp.float32),
    )(h, w1, b1.reshape(1, D), w2, b2.reshape(1, D), w3t)
    return out[0, :1] + b3


def _pad_planes(a):
    return jnp.pad(a.T, ((0, 0), (0, NP - N)))


def _gat_layer(h, srcb, dstb, src_ids, dst, idxS, idxD, ea, loop_attr,
               lin_w, att_src, att_dst, lin_edge_w, att_edge, bias):
    xh_flat, a_src, a_dst = _attn_proj(h, lin_w, att_src, att_dst)
    xh = xh_flat.reshape(N, H, D)
    t = a_dst

    V = (lin_edge_w.reshape(3, H, D) * att_edge.reshape(1, H, D)).sum(axis=2)

    g = ea @ V
    g_flat = jnp.concatenate(
        [g.reshape(-1), jnp.full(4 * (EPAD - E), -1e30, jnp.float32)])

    def padp(a):
        return jnp.concatenate([a.reshape(-1), jnp.zeros(ANP - 4 * N, jnp.float32)])

    le_flat = _sc_le(idxS, idxD, g_flat, padp(a_src), padp(t))

    le = le_flat[:4 * E].reshape(E, H)
    le_max = jnp.maximum(jax.ops.segment_max(le, dst, num_segments=N), -1e30)

    q_loop = a_src + loop_attr @ V
    alpha_l = jax.nn.leaky_relu(q_loop + t, negative_slope=0.2)
    amax = jnp.maximum(le_max, alpha_l)

    w_flat = _sc_w(idxD, le_flat, padp(amax))

    ex_l = jnp.exp(alpha_l - amax)

    dparts = _sc_degsum(idxD, w_flat.reshape(-1, CH))
    denom = (dparts[0, :4 * N] + dparts[1, :4 * N]).reshape(N, 4) + ex_l

    w2 = w_flat.reshape(EPAD, 4)
    out_cols = []
    for hp in range(2):
        xhp = xh_flat[:, hp * 128:(hp + 1) * 128]
        w0 = w2[:, 2 * hp]
        w1 = w2[:, 2 * hp + 1]
        rows = []
        for r in range(4):
            part = _sc_msg[r](srcb, dstb, xhp, w0, w1)
            rows.append((part[0] + part[1])[:QR])
        out_cols.append(jnp.concatenate(rows, axis=0)[:N])
    out_un = jnp.concatenate(out_cols, axis=1).reshape(N, H, D)
    out_un = out_un + ex_l[:, :, None] * xh
    out = out_un / (denom[:, :, None] + 1e-16)
    return out.mean(axis=1) + bias


def kernel(x, edge_index, edge_attr, W_in, b_in, lin_w0, att_src0, att_dst0,
           lin_edge_w0, att_edge0, bias0, lin_w1, att_src1, att_dst1,
           lin_edge_w1, att_edge1, bias1, fc1_w, fc1_b, fc2_w, fc2_b, fc3_w, fc3_b):
    src, dst = edge_index[0], edge_index[1]
    ea = edge_attr.at[:, 2].set(1000000.0 / edge_attr[:, 2])

    idxA = (4 * dst[:, None] + jnp.arange(4, dtype=jnp.int32)[None, :]).reshape(-1)
    idxA = jnp.concatenate([idxA, jnp.zeros(4 * (EPAD - E), jnp.int32)])
    datA = jnp.concatenate([jnp.ones((E, 1), jnp.float32), ea], axis=1).reshape(-1)
    datA = jnp.concatenate([datA, jnp.zeros(4 * (EPAD - E), jnp.float32)])
    parts = _sc_degsum(idxA.reshape(-1, CH), datA.reshape(-1, CH))
    merged = (parts[0, :4 * N] + parts[1, :4 * N]).reshape(N, 4)
    deg = merged[:, 0]
    loop_attr = merged[:, 1:4] / jnp.maximum(deg, 1.0)[:, None]

    srcb = jnp.concatenate([src, jnp.zeros(EPAD - E, jnp.int32)]).reshape(EPAD // CH, CH)
    dstb = jnp.concatenate([dst, jnp.zeros(EPAD - E, jnp.int32)]).reshape(EPAD // CH, CH)
    idxD = idxA.reshape(-1, CH)
    idxS = (4 * src[:, None] + jnp.arange(4, dtype=jnp.int32)[None, :]).reshape(-1)
    idxS = jnp.concatenate([idxS, jnp.zeros(4 * (EPAD - E), jnp.int32)]).reshape(-1, CH)

    h = _inproj(x, W_in, b_in)
    h = _gelu(_gat_layer(h, srcb, dstb, src, dst, idxS, idxD, ea, loop_attr,
                         lin_w0, att_src0, att_dst0, lin_edge_w0, att_edge0, bias0))
    identity = h
    h = _gat_layer(h, srcb, dstb, src, dst, idxS, idxD, ea, loop_attr,
                   lin_w1, att_src1, att_dst1, lin_edge_w1, att_edge1, bias1)
    h = _gelu(h + identity)
    return _mlp_min(h, fc1_w, fc1_b, fc2_w, fc2_b, fc3_w, fc3_b)
